# baseline (speedup 1.0000x reference)
import functools
import math

import jax
import jax.numpy as jnp
from jax import lax
from jax.experimental import pallas as pl
from jax.experimental.pallas import tpu as pltpu
from jax.experimental.pallas import tpu_sc as plsc

EMB_DIM = 164
SEQ_LEN = 20
SCALE = math.sqrt(float(EMB_DIM))
LANES = 16

NUM_CORES = 2
NUM_SUBCORES = 16
NUM_WORKERS = NUM_CORES * NUM_SUBCORES

SEQS_PER_CHUNK = 4
ROWS_PER_CHUNK = SEQS_PER_CHUNK * SEQ_LEN


def _pe_table():
    position = jnp.arange(0, SEQ_LEN, dtype=jnp.float32)[:, None]
    div_term = jnp.exp(
        jnp.arange(0, EMB_DIM, 2, dtype=jnp.float32) * -(math.log(10000.0) / EMB_DIM)
    )
    angles = position * div_term
    pe = jnp.zeros((SEQ_LEN, EMB_DIM), dtype=jnp.float32)
    pe = pe.at[:, 0::2].set(jnp.sin(angles))
    pe = pe.at[:, 1::2].set(jnp.cos(angles))
    return pe


def kernel(input_ids, embedding_weight):
    n_seq, seq_len = input_ids.shape
    total = n_seq * seq_len
    rows_per_worker = total // NUM_WORKERS
    n_chunks = rows_per_worker // ROWS_PER_CHUNK

    ids2 = input_ids.astype(jnp.int32).reshape(NUM_WORKERS, rows_per_worker)
    pe = _pe_table()

    mesh = plsc.VectorSubcoreMesh(core_axis_name="c", subcore_axis_name="s")

    @functools.partial(
        pl.kernel,
        mesh=mesh,
        out_type=jax.ShapeDtypeStruct((total, EMB_DIM), jnp.float32),
        compiler_params=pltpu.CompilerParams(use_tc_tiling_on_sc=False),
        scratch_types=[
            pltpu.VMEM((rows_per_worker,), jnp.int32),
            pltpu.VMEM((SEQ_LEN, EMB_DIM), jnp.float32),
            pltpu.VMEM((ROWS_PER_CHUNK, EMB_DIM), jnp.float32),
            pltpu.VMEM((ROWS_PER_CHUNK, EMB_DIM), jnp.float32),
            pltpu.SemaphoreType.DMA,
        ],
    )
    def _emb(ids_hbm, table_hbm, pe_hbm, out_hbm, idx_v, pe_v, in_v, out_v, sem):
        wid = lax.axis_index("s") * NUM_CORES + lax.axis_index("c")
        base = wid * rows_per_worker
        pltpu.sync_copy(ids_hbm.at[wid], idx_v)
        pltpu.sync_copy(pe_hbm, pe_v)

        def chunk_body(c, carry):
            copies = []
            for g in range(ROWS_PER_CHUNK // LANES):
                vec = idx_v[pl.ds(c * ROWS_PER_CHUNK + g * LANES, LANES)]
                for l in range(LANES):
                    copies.append(
                        pltpu.async_copy(
                            table_hbm.at[vec[l]], in_v.at[g * LANES + l], sem
                        )
                    )
            for cp in copies:
                cp.wait()

            def seq_body(s, carry2):
                for p in range(SEQ_LEN):
                    r = s * SEQ_LEN + p
                    for k in range(EMB_DIM // LANES):
                        sl = pl.ds(k * LANES, LANES)
                        out_v[r, sl] = in_v[r, sl] * SCALE + pe_v[p, sl]
                    sl = pl.ds(EMB_DIM - LANES, LANES)
                    out_v[r, sl] = in_v[r, sl] * SCALE + pe_v[p, sl]
                return carry2

            lax.fori_loop(0, SEQS_PER_CHUNK, seq_body, 0)
            pltpu.sync_copy(
                out_v, out_hbm.at[pl.ds(base + c * ROWS_PER_CHUNK, ROWS_PER_CHUNK)]
            )
            return carry

        lax.fori_loop(0, n_chunks, chunk_body, 0)

    out = _emb(ids2, embedding_weight, pe)
    return out.reshape(n_seq, seq_len, EMB_DIM)


# speedup vs baseline: 1.0219x; 1.0219x over previous
import functools
import math

import jax
import jax.numpy as jnp
from jax import lax
from jax.experimental import pallas as pl
from jax.experimental.pallas import tpu as pltpu
from jax.experimental.pallas import tpu_sc as plsc

EMB_DIM = 164
SEQ_LEN = 20
SCALE = math.sqrt(float(EMB_DIM))
LANES = 16

NUM_CORES = 2
NUM_SUBCORES = 16
NUM_WORKERS = NUM_CORES * NUM_SUBCORES

SEQS_PER_CHUNK = 8
ROWS_PER_CHUNK = SEQS_PER_CHUNK * SEQ_LEN


def _pe_table():
    position = jnp.arange(0, SEQ_LEN, dtype=jnp.float32)[:, None]
    div_term = jnp.exp(
        jnp.arange(0, EMB_DIM, 2, dtype=jnp.float32) * -(math.log(10000.0) / EMB_DIM)
    )
    angles = position * div_term
    pe = jnp.zeros((SEQ_LEN, EMB_DIM), dtype=jnp.float32)
    pe = pe.at[:, 0::2].set(jnp.sin(angles))
    pe = pe.at[:, 1::2].set(jnp.cos(angles))
    return pe


def kernel(input_ids, embedding_weight):
    n_seq, seq_len = input_ids.shape
    total = n_seq * seq_len
    rows_per_worker = total // NUM_WORKERS
    n_chunks = rows_per_worker // ROWS_PER_CHUNK

    ids2 = input_ids.astype(jnp.int32).reshape(NUM_WORKERS, rows_per_worker)
    pe = _pe_table()

    mesh = plsc.VectorSubcoreMesh(core_axis_name="c", subcore_axis_name="s")

    @functools.partial(
        pl.kernel,
        mesh=mesh,
        out_type=jax.ShapeDtypeStruct((total, EMB_DIM), jnp.float32),
        compiler_params=pltpu.CompilerParams(use_tc_tiling_on_sc=False),
        scratch_types=[
            pltpu.VMEM((rows_per_worker,), jnp.int32),
            pltpu.VMEM((SEQ_LEN, EMB_DIM), jnp.float32),
            pltpu.VMEM((ROWS_PER_CHUNK, EMB_DIM), jnp.float32),
            pltpu.VMEM((ROWS_PER_CHUNK, EMB_DIM), jnp.float32),
            pltpu.SemaphoreType.DMA,
            pltpu.SemaphoreType.DMA,
        ],
    )
    def _emb(ids_hbm, table_hbm, pe_hbm, out_hbm, idx_v, pe_v, buf0, buf1, s0, s1):
        wid = lax.axis_index("s") * NUM_CORES + lax.axis_index("c")
        base = wid * rows_per_worker
        pltpu.sync_copy(ids_hbm.at[wid], idx_v)
        pltpu.sync_copy(pe_hbm, pe_v)

        bufs = (buf0, buf1)
        sems = (s0, s1)

        def issue_gather(c, buf, sem):
            for g in range(ROWS_PER_CHUNK // LANES):
                vec = idx_v[pl.ds(c * ROWS_PER_CHUNK + g * LANES, LANES)]
                for l in range(LANES):
                    pltpu.async_copy(table_hbm.at[vec[l]], buf.at[g * LANES + l], sem)

        def wait_gather(buf, sem):
            pltpu.make_async_copy(
                out_hbm.at[pl.ds(0, ROWS_PER_CHUNK)], buf, sem
            ).wait()

        def compute(buf):
            def seq_body(s, carry):
                for p in range(SEQ_LEN):
                    r = s * SEQ_LEN + p
                    slices = [
                        pl.ds(k * LANES, LANES) for k in range(EMB_DIM // LANES)
                    ] + [pl.ds(EMB_DIM - LANES, LANES)]
                    vals = [buf[r, sl] * SCALE + pe_v[p, sl] for sl in slices]
                    for sl, v in zip(slices, vals):
                        buf[r, sl] = v
                return carry

            lax.fori_loop(0, SEQS_PER_CHUNK, seq_body, 0)

        issue_gather(0, buf0, s0)

        def loop_body(c0, carry):
            for b in range(2):
                c = c0 + b
                nxt = c + 1

                @pl.when(nxt < n_chunks)
                def _():
                    issue_gather(nxt, bufs[1 - b], sems[1 - b])

                wait_gather(bufs[b], sems[b])
                compute(bufs[b])
                pltpu.sync_copy(
                    bufs[b],
                    out_hbm.at[pl.ds(base + c * ROWS_PER_CHUNK, ROWS_PER_CHUNK)],
                )
            return carry

        lax.fori_loop(0, n_chunks // 2, lambda i, cr: loop_body(i * 2, cr), 0)

    out = _emb(ids2, embedding_weight, pe)
    return out.reshape(n_seq, seq_len, EMB_DIM)


# speedup vs baseline: 6.1928x; 6.0602x over previous
import functools
import math

import jax
import jax.numpy as jnp
from jax import lax
from jax.experimental import pallas as pl
from jax.experimental.pallas import tpu as pltpu
from jax.experimental.pallas import tpu_sc as plsc

EMB_DIM = 164
SEQ_LEN = 20
SCALE = math.sqrt(float(EMB_DIM))
LANES = 16

NUM_CORES = 2
NUM_SUBCORES = 16
NUM_WORKERS = NUM_CORES * NUM_SUBCORES

SEQS_PER_CHUNK = 8
ROWS_PER_CHUNK = SEQS_PER_CHUNK * SEQ_LEN


def _pe_table():
    position = jnp.arange(0, SEQ_LEN, dtype=jnp.float32)[:, None]
    div_term = jnp.exp(
        jnp.arange(0, EMB_DIM, 2, dtype=jnp.float32) * -(math.log(10000.0) / EMB_DIM)
    )
    angles = position * div_term
    pe = jnp.zeros((SEQ_LEN, EMB_DIM), dtype=jnp.float32)
    pe = pe.at[:, 0::2].set(jnp.sin(angles))
    pe = pe.at[:, 1::2].set(jnp.cos(angles))
    return pe


def kernel(input_ids, embedding_weight):
    n_seq, seq_len = input_ids.shape
    total = n_seq * seq_len
    rows_per_worker = total // NUM_WORKERS
    n_chunks = rows_per_worker // ROWS_PER_CHUNK

    ids2 = input_ids.astype(jnp.int32).reshape(NUM_WORKERS, rows_per_worker)
    pe = _pe_table()

    mesh = plsc.VectorSubcoreMesh(core_axis_name="c", subcore_axis_name="s")

    @functools.partial(
        pl.kernel,
        mesh=mesh,
        out_type=jax.ShapeDtypeStruct((total, EMB_DIM), jnp.float32),
        compiler_params=pltpu.CompilerParams(use_tc_tiling_on_sc=True),
        scratch_types=[
            pltpu.VMEM((rows_per_worker,), jnp.int32),
            pltpu.VMEM((SEQ_LEN, EMB_DIM), jnp.float32),
            pltpu.VMEM((ROWS_PER_CHUNK, EMB_DIM), jnp.float32),
            pltpu.VMEM((ROWS_PER_CHUNK, EMB_DIM), jnp.float32),
            pltpu.SemaphoreType.DMA,
            pltpu.SemaphoreType.DMA,
        ],
    )
    def _emb(ids_hbm, table_hbm, pe_hbm, out_hbm, idx_v, pe_v, buf0, buf1, s0, s1):
        wid = lax.axis_index("s") * NUM_CORES + lax.axis_index("c")
        base = wid * rows_per_worker
        pltpu.sync_copy(ids_hbm.at[wid], idx_v)
        pltpu.sync_copy(pe_hbm, pe_v)

        bufs = (buf0, buf1)
        sems = (s0, s1)

        def issue_gather(c, buf, sem):
            for g in range(ROWS_PER_CHUNK // LANES):
                vec = idx_v[pl.ds(c * ROWS_PER_CHUNK + g * LANES, LANES)]
                for l in range(LANES):
                    pltpu.async_copy(table_hbm.at[vec[l]], buf.at[g * LANES + l], sem)

        def wait_gather(buf, sem):
            pltpu.make_async_copy(
                out_hbm.at[pl.ds(0, ROWS_PER_CHUNK)], buf, sem
            ).wait()

        def compute(buf):
            def seq_body(s, carry):
                for p in range(SEQ_LEN):
                    r = s * SEQ_LEN + p
                    slices = [
                        pl.ds(k * LANES, LANES) for k in range(EMB_DIM // LANES)
                    ] + [pl.ds(EMB_DIM - LANES, LANES)]
                    vals = [buf[r, sl] * SCALE + pe_v[p, sl] for sl in slices]
                    for sl, v in zip(slices, vals):
                        buf[r, sl] = v
                return carry

            lax.fori_loop(0, SEQS_PER_CHUNK, seq_body, 0)

        issue_gather(0, buf0, s0)

        def loop_body(c0, carry):
            for b in range(2):
                c = c0 + b
                nxt = c + 1

                @pl.when(nxt < n_chunks)
                def _():
                    issue_gather(nxt, bufs[1 - b], sems[1 - b])

                wait_gather(bufs[b], sems[b])
                compute(bufs[b])
                pltpu.sync_copy(
                    bufs[b],
                    out_hbm.at[pl.ds(base + c * ROWS_PER_CHUNK, ROWS_PER_CHUNK)],
                )
            return carry

        lax.fori_loop(0, n_chunks // 2, lambda i, cr: loop_body(i * 2, cr), 0)

    out = _emb(ids2, embedding_weight, pe)
    return out.reshape(n_seq, seq_len, EMB_DIM)


# speedup vs baseline: 6.6342x; 1.0713x over previous
import functools
import math

import jax
import jax.numpy as jnp
from jax import lax
from jax.experimental import pallas as pl
from jax.experimental.pallas import tpu as pltpu
from jax.experimental.pallas import tpu_sc as plsc

EMB_DIM = 164
SEQ_LEN = 20
SCALE = math.sqrt(float(EMB_DIM))
LANES = 16

NUM_CORES = 2
NUM_SUBCORES = 16
NUM_WORKERS = NUM_CORES * NUM_SUBCORES

SEQS_PER_CHUNK = 8
ROWS_PER_CHUNK = SEQS_PER_CHUNK * SEQ_LEN


def _pe_table():
    position = jnp.arange(0, SEQ_LEN, dtype=jnp.float32)[:, None]
    div_term = jnp.exp(
        jnp.arange(0, EMB_DIM, 2, dtype=jnp.float32) * -(math.log(10000.0) / EMB_DIM)
    )
    angles = position * div_term
    pe = jnp.zeros((SEQ_LEN, EMB_DIM), dtype=jnp.float32)
    pe = pe.at[:, 0::2].set(jnp.sin(angles))
    pe = pe.at[:, 1::2].set(jnp.cos(angles))
    return pe


def kernel(input_ids, embedding_weight):
    n_seq, seq_len = input_ids.shape
    total = n_seq * seq_len
    rows_per_worker = total // NUM_WORKERS
    n_chunks = rows_per_worker // ROWS_PER_CHUNK

    ids2 = input_ids.astype(jnp.int32).reshape(NUM_WORKERS, rows_per_worker)
    pe = _pe_table()

    mesh = plsc.VectorSubcoreMesh(core_axis_name="c", subcore_axis_name="s")

    @functools.partial(
        pl.kernel,
        mesh=mesh,
        out_type=jax.ShapeDtypeStruct((n_seq, SEQ_LEN, EMB_DIM), jnp.float32),
        compiler_params=pltpu.CompilerParams(use_tc_tiling_on_sc=True),
        scratch_types=[
            pltpu.VMEM((rows_per_worker,), jnp.int32),
            pltpu.VMEM((SEQ_LEN, EMB_DIM), jnp.float32),
            pltpu.VMEM((SEQS_PER_CHUNK, SEQ_LEN, EMB_DIM), jnp.float32),
            pltpu.VMEM((SEQS_PER_CHUNK, SEQ_LEN, EMB_DIM), jnp.float32),
            pltpu.SemaphoreType.DMA,
            pltpu.SemaphoreType.DMA,
        ],
    )
    def _emb(ids_hbm, table_hbm, pe_hbm, out_hbm, idx_v, pe_v, buf0, buf1, s0, s1):
        wid = lax.axis_index("s") * NUM_CORES + lax.axis_index("c")
        seq_base = wid * (rows_per_worker // SEQ_LEN)
        pltpu.sync_copy(ids_hbm.at[wid], idx_v)
        pltpu.sync_copy(pe_hbm, pe_v)

        bufs = (buf0, buf1)
        sems = (s0, s1)

        def issue_gather(c, buf, sem):
            for g in range(ROWS_PER_CHUNK // LANES):
                vec = idx_v[pl.ds(c * ROWS_PER_CHUNK + g * LANES, LANES)]
                for l in range(LANES):
                    j = g * LANES + l
                    pltpu.async_copy(
                        table_hbm.at[vec[l]], buf.at[j // SEQ_LEN, j % SEQ_LEN], sem
                    )

        def wait_gather(buf, sem):
            for q in range(SEQS_PER_CHUNK):
                for p in range(SEQ_LEN):
                    pltpu.make_async_copy(
                        table_hbm.at[0], buf.at[q, p], sem
                    ).wait()

        def compute(buf):
            def seq_body(q, carry):
                for p in range(SEQ_LEN):
                    slices = [
                        pl.ds(k * LANES, LANES) for k in range(EMB_DIM // LANES)
                    ] + [pl.ds(EMB_DIM - LANES, LANES)]
                    vals = [buf[q, p, sl] * SCALE + pe_v[p, sl] for sl in slices]
                    for sl, v in zip(slices, vals):
                        buf[q, p, sl] = v
                return carry

            lax.fori_loop(0, SEQS_PER_CHUNK, seq_body, 0)

        issue_gather(0, buf0, s0)

        def loop_body(c0, carry):
            for b in range(2):
                c = c0 + b
                nxt = c + 1

                @pl.when(nxt < n_chunks)
                def _():
                    issue_gather(nxt, bufs[1 - b], sems[1 - b])

                wait_gather(bufs[b], sems[b])
                compute(bufs[b])
                pltpu.sync_copy(
                    bufs[b],
                    out_hbm.at[pl.ds(seq_base + c * SEQS_PER_CHUNK, SEQS_PER_CHUNK)],
                )
            return carry

        lax.fori_loop(0, n_chunks // 2, lambda i, cr: loop_body(i * 2, cr), 0)

    return _emb(ids2, embedding_weight, pe)
